# baseline (speedup 1.0000x reference)
import functools

import jax
import jax.numpy as jnp
from jax import lax
from jax.experimental import pallas as pl
from jax.experimental.pallas import tpu as pltpu
from jax.experimental.pallas import tpu_sc as plsc

B = 4096
D = 64
L = 5
NC = 2
NS = 16
NW = NC * NS
BPW = B // NW
CH = 16
LANES = 16


def _body(uid_hbm, iid_hbm, ue_hbm, ie_hbm, ib_hbm, ub_hbm, out_hbm,
          uid_v, iid_v, ueq_v, ieq_v, ibq_v, ubq_v,
          u_v0, u_v1, v_v0, v_v1, ib_v, ub_v, out_v, sem):
    u_bufs = (u_v0, u_v1)
    v_bufs = (v_v0, v_v1)
    wid = lax.axis_index("s") * NC + lax.axis_index("c")
    base = wid * BPW

    pltpu.sync_copy(uid_hbm.at[pl.ds(base, BPW)], uid_v)
    pltpu.sync_copy(iid_hbm.at[pl.ds(base, BPW)], iid_v)

    for c in range(BPW // LANES):
        sl = pl.ds(c * LANES, LANES)
        ueq_v[sl] = lax.shift_right_logical(uid_v[sl], 4)
        ieq_v[sl] = lax.shift_right_logical(iid_v[sl], 4)
        ubq_v[sl] = lax.shift_right_logical(uid_v[sl], 5)
        ibq_v[sl] = lax.shift_right_logical(iid_v[sl], 7)

    cp_ib = pltpu.async_copy(ib_hbm.at[ibq_v], ib_v, sem)
    cp_ub = pltpu.async_copy(ub_hbm.at[ubq_v], ub_v, sem)

    lane = lax.iota(jnp.int32, LANES)
    one = jnp.float32(1.0)

    def _exp(x):
        t = x * jnp.float32(1.4426950408889634)
        n = t.astype(jnp.int32)
        g = (t - n.astype(jnp.float32)) * jnp.float32(0.6931471805599453)
        p = one + g * jnp.float32(1 / 7.0)
        for r in (6.0, 5.0, 4.0, 3.0, 2.0, 1.0):
            p = one + g * jnp.float32(1 / r) * p
        scale = lax.bitcast_convert_type(
            lax.shift_left(n + 127, jnp.full((LANES,), 23, jnp.int32)),
            jnp.float32)
        return p * scale

    cp_u0 = pltpu.async_copy(ue_hbm.at[ueq_v.at[pl.ds(0, CH)]], u_v0, sem)
    cp_v0 = pltpu.async_copy(ie_hbm.at[ieq_v.at[pl.ds(0, CH)]], v_v0, sem)
    cp_u0.wait()
    cp_v0.wait()
    cp_ib.wait()
    cp_ub.wait()

    for ch in range(BPW // CH):
        u_v = u_bufs[ch % 2]
        v_v = v_bufs[ch % 2]
        if ch + 1 < BPW // CH:
            cp_u = pltpu.async_copy(
                ue_hbm.at[ueq_v.at[pl.ds((ch + 1) * CH, CH)]],
                u_bufs[(ch + 1) % 2], sem)
            cp_v = pltpu.async_copy(
                ie_hbm.at[ieq_v.at[pl.ds((ch + 1) * CH, CH)]],
                v_bufs[(ch + 1) % 2], sem)
        for g in range(CH // LANES):
            e0 = ch * CH + g * LANES
            sl = pl.ds(e0, LANES)
            rows = e0 + lane
            jvec = g * LANES + lane
            ur16 = jnp.bitwise_and(uid_v[sl], 15) * D
            vr16 = jnp.bitwise_and(iid_v[sl], 15) * D
            c127 = jnp.full((LANES,), 127, jnp.int32)

            def dot_step(d, acc):
                ut = ur16 + d
                vt = vr16 + d
                ug = plsc.load_gather(
                    u_v, [jvec, lax.shift_right_logical(ut, 7),
                          jnp.bitwise_and(ut, c127)])
                vg = plsc.load_gather(
                    v_v, [jvec, lax.shift_right_logical(vt, 7),
                          jnp.bitwise_and(vt, c127)])
                return acc + ug * vg

            y = lax.fori_loop(0, D, dot_step,
                              jnp.zeros((LANES,), jnp.float32))
            y = y + plsc.load_gather(
                ib_v, [rows, jnp.bitwise_and(iid_v[sl], 127)])

            ub_col = jnp.bitwise_and(uid_v[sl], 31) * (L - 1)
            cut = plsc.load_gather(ub_v, [rows, ub_col])
            d_prev = one / (one + _exp(y - cut))
            plsc.store_scatter(out_v,
                               [rows, jnp.zeros((LANES,), jnp.int32)], d_prev)
            for k in range(1, L - 1):
                cut = cut + _exp(plsc.load_gather(ub_v, [rows, ub_col + k]))
                d_k = one / (one + _exp(y - cut))
                plsc.store_scatter(
                    out_v, [rows, jnp.full((LANES,), k, jnp.int32)],
                    d_k - d_prev)
                d_prev = d_k
            plsc.store_scatter(
                out_v, [rows, jnp.full((LANES,), L - 1, jnp.int32)],
                one - d_prev)
        if ch + 1 < BPW // CH:
            cp_u.wait()
            cp_v.wait()

    pltpu.sync_copy(out_v, out_hbm.at[pl.ds(base, BPW)])


@functools.partial(
    pl.kernel,
    mesh=plsc.VectorSubcoreMesh(core_axis_name="c", subcore_axis_name="s"),
    out_type=jax.ShapeDtypeStruct((B, L), jnp.float32),
    compiler_params=pltpu.CompilerParams(
        needs_layout_passes=False, use_tc_tiling_on_sc=True),
    scratch_types=[
        pltpu.VMEM((BPW,), jnp.int32),
        pltpu.VMEM((BPW,), jnp.int32),
        pltpu.VMEM((BPW,), jnp.int32),
        pltpu.VMEM((BPW,), jnp.int32),
        pltpu.VMEM((BPW,), jnp.int32),
        pltpu.VMEM((BPW,), jnp.int32),
        pltpu.VMEM((CH, 8, 128), jnp.float32),
        pltpu.VMEM((CH, 8, 128), jnp.float32),
        pltpu.VMEM((CH, 8, 128), jnp.float32),
        pltpu.VMEM((CH, 8, 128), jnp.float32),
        pltpu.VMEM((BPW, 128), jnp.float32),
        pltpu.VMEM((BPW, 128), jnp.float32),
        pltpu.VMEM((BPW, L), jnp.float32),
        pltpu.SemaphoreType.DMA,
    ],
)
def _ordrec_sc(*args):
    _body(*args)


def kernel(user_ids, item_ids, user_embeddings, item_embeddings,
           item_biases, user_betas):
    n_items = item_biases.shape[0]
    ue3 = user_embeddings.reshape(-1, 8, 128)
    ie3 = item_embeddings.reshape(-1, 8, 128)
    ub2 = user_betas.reshape(-1, 128)
    pad = (-n_items) % 128
    ib2 = jnp.pad(item_biases.reshape(-1), (0, pad)).reshape(-1, 128)
    return _ordrec_sc(user_ids.astype(jnp.int32), item_ids.astype(jnp.int32),
                      ue3, ie3, ib2, ub2)


# speedup vs baseline: 1.2815x; 1.2815x over previous
import functools

import jax
import jax.numpy as jnp
from jax import lax
from jax.experimental import pallas as pl
from jax.experimental.pallas import tpu as pltpu
from jax.experimental.pallas import tpu_sc as plsc

B = 4096
D = 64
L = 5
NC = 2
NS = 16
NW = NC * NS
BPW = B // NW
CH = 32
LANES = 16
NG = BPW // LANES


def _body(uid_hbm, iid_hbm, ue_hbm, ie_hbm, ib_hbm,
          ub0_hbm, ub1_hbm, ub2_hbm, ub3_hbm, out_hbm,
          uid_v, iid_v, ueq_v, ieq_v, ibq_v, ubq_v,
          u_v, v_v, big_v, ib_s, ub0_s, ub1_s, ub2_s, ub3_s, out_v, sem):
    wid = lax.axis_index("s") * NC + lax.axis_index("c")
    base = wid * BPW

    pltpu.sync_copy(uid_hbm.at[pl.ds(base, BPW)], uid_v)
    pltpu.sync_copy(iid_hbm.at[pl.ds(base, BPW)], iid_v)

    for c in range(NG):
        sl = pl.ds(c * LANES, LANES)
        ueq_v[sl] = lax.shift_right_logical(uid_v[sl], 4)
        ieq_v[sl] = lax.shift_right_logical(iid_v[sl], 4)
        ubq_v[sl] = lax.shift_right_logical(uid_v[sl], 7)
        ibq_v[sl] = lax.shift_right_logical(iid_v[sl], 7)

    lane = lax.iota(jnp.int32, LANES)
    one = jnp.float32(1.0)
    c127 = jnp.full((LANES,), 127, jnp.int32)

    for src, qv, idv, dst in (
            (ib_hbm, ibq_v, iid_v, ib_s),
            (ub0_hbm, ubq_v, uid_v, ub0_s),
            (ub1_hbm, ubq_v, uid_v, ub1_s),
            (ub2_hbm, ubq_v, uid_v, ub2_s),
            (ub3_hbm, ubq_v, uid_v, ub3_s)):
        pltpu.async_copy(src.at[qv], big_v, sem).wait()
        for c in range(NG):
            sl = pl.ds(c * LANES, LANES)
            rows = c * LANES + lane
            dst[sl] = plsc.load_gather(
                big_v, [rows, jnp.bitwise_and(idv[sl], c127)])

    def _exp(x):
        t = x * jnp.float32(1.4426950408889634)
        n = t.astype(jnp.int32)
        g = (t - n.astype(jnp.float32)) * jnp.float32(0.6931471805599453)
        p = one + g * jnp.float32(1 / 7.0)
        for r in (6.0, 5.0, 4.0, 3.0, 2.0, 1.0):
            p = one + g * jnp.float32(1 / r) * p
        scale = lax.bitcast_convert_type(
            lax.shift_left(n + 127, jnp.full((LANES,), 23, jnp.int32)),
            jnp.float32)
        return p * scale

    for ch in range(BPW // CH):
        cp_u = pltpu.async_copy(
            ue_hbm.at[ueq_v.at[pl.ds(ch * CH, CH)]], u_v, sem)
        cp_v = pltpu.async_copy(
            ie_hbm.at[ieq_v.at[pl.ds(ch * CH, CH)]], v_v, sem)
        cp_u.wait()
        cp_v.wait()
        for g in range(CH // LANES):
            e0 = ch * CH + g * LANES
            sl = pl.ds(e0, LANES)
            rows = e0 + lane
            jvec = g * LANES + lane
            ur16 = jnp.bitwise_and(uid_v[sl], 15) * D
            vr16 = jnp.bitwise_and(iid_v[sl], 15) * D

            def dot_step(d4, acc):
                d = d4 * 4
                for s in range(4):
                    ut = ur16 + (d + s)
                    vt = vr16 + (d + s)
                    ug = plsc.load_gather(
                        u_v, [jvec, lax.shift_right_logical(ut, 7),
                              jnp.bitwise_and(ut, c127)])
                    vg = plsc.load_gather(
                        v_v, [jvec, lax.shift_right_logical(vt, 7),
                              jnp.bitwise_and(vt, c127)])
                    acc = acc + ug * vg
                return acc

            y = lax.fori_loop(0, D // 4, dot_step,
                              jnp.zeros((LANES,), jnp.float32))
            y = y + ib_s[sl]

            cut = ub0_s[sl]
            d_prev = one / (one + _exp(y - cut))
            plsc.store_scatter(out_v,
                               [rows, jnp.zeros((LANES,), jnp.int32)], d_prev)
            for k, ub_s in ((1, ub1_s), (2, ub2_s), (3, ub3_s)):
                cut = cut + _exp(ub_s[sl])
                d_k = one / (one + _exp(y - cut))
                plsc.store_scatter(
                    out_v, [rows, jnp.full((LANES,), k, jnp.int32)],
                    d_k - d_prev)
                d_prev = d_k
            plsc.store_scatter(
                out_v, [rows, jnp.full((LANES,), L - 1, jnp.int32)],
                one - d_prev)

    pltpu.sync_copy(out_v, out_hbm.at[pl.ds(base, BPW)])


@functools.partial(
    pl.kernel,
    mesh=plsc.VectorSubcoreMesh(core_axis_name="c", subcore_axis_name="s"),
    out_type=jax.ShapeDtypeStruct((B, L), jnp.float32),
    compiler_params=pltpu.CompilerParams(
        needs_layout_passes=False, use_tc_tiling_on_sc=True),
    scratch_types=[
        pltpu.VMEM((BPW,), jnp.int32),
        pltpu.VMEM((BPW,), jnp.int32),
        pltpu.VMEM((BPW,), jnp.int32),
        pltpu.VMEM((BPW,), jnp.int32),
        pltpu.VMEM((BPW,), jnp.int32),
        pltpu.VMEM((BPW,), jnp.int32),
        pltpu.VMEM((CH, 8, 128), jnp.float32),
        pltpu.VMEM((CH, 8, 128), jnp.float32),
        pltpu.VMEM((BPW, 128), jnp.float32),
        pltpu.VMEM((BPW,), jnp.float32),
        pltpu.VMEM((BPW,), jnp.float32),
        pltpu.VMEM((BPW,), jnp.float32),
        pltpu.VMEM((BPW,), jnp.float32),
        pltpu.VMEM((BPW,), jnp.float32),
        pltpu.VMEM((BPW, L), jnp.float32),
        pltpu.SemaphoreType.DMA,
    ],
)
def _ordrec_sc(*args):
    _body(*args)


def _colpad(col):
    pad = (-col.shape[0]) % 128
    return jnp.pad(col, (0, pad)).reshape(-1, 128)


def kernel(user_ids, item_ids, user_embeddings, item_embeddings,
           item_biases, user_betas):
    ue3 = user_embeddings.reshape(-1, 8, 128)
    ie3 = item_embeddings.reshape(-1, 8, 128)
    ib2 = _colpad(item_biases[:, 0])
    ub = [_colpad(user_betas[:, k]) for k in range(L - 1)]
    return _ordrec_sc(user_ids.astype(jnp.int32), item_ids.astype(jnp.int32),
                      ue3, ie3, ib2, ub[0], ub[1], ub[2], ub[3])


# speedup vs baseline: 1.3541x; 1.0566x over previous
import functools

import jax
import jax.numpy as jnp
from jax import lax
from jax.experimental import pallas as pl
from jax.experimental.pallas import tpu as pltpu
from jax.experimental.pallas import tpu_sc as plsc

B = 4096
D = 64
L = 5
NC = 2
NS = 16
NW = NC * NS
BPW = B // NW
CH = 16
LANES = 16
NG = BPW // LANES


def _body(uid_hbm, iid_hbm, ue_hbm, ie_hbm, ib_hbm,
          ub0_hbm, ub1_hbm, ub2_hbm, ub3_hbm, out_hbm,
          uid_v, iid_v, ueq_v, ieq_v, ibq_v, ubq_v,
          u_v0, u_v1, v_v0, v_v1, big_a, big_b, ib_s, ub0_s, ub1_s, ub2_s,
          ub3_s, out_v, sem):
    wid = lax.axis_index("s") * NC + lax.axis_index("c")
    base = wid * BPW

    pltpu.sync_copy(uid_hbm.at[pl.ds(base, BPW)], uid_v)
    pltpu.sync_copy(iid_hbm.at[pl.ds(base, BPW)], iid_v)

    for c in range(NG):
        sl = pl.ds(c * LANES, LANES)
        ueq_v[sl] = lax.shift_right_logical(uid_v[sl], 4)
        ieq_v[sl] = lax.shift_right_logical(iid_v[sl], 4)
        ubq_v[sl] = lax.shift_right_logical(uid_v[sl], 7)
        ibq_v[sl] = lax.shift_right_logical(iid_v[sl], 7)

    lane = lax.iota(jnp.int32, LANES)
    one = jnp.float32(1.0)
    c127 = jnp.full((LANES,), 127, jnp.int32)

    stages = [
        (ib_hbm, ibq_v, iid_v, ib_s),
        (ub0_hbm, ubq_v, uid_v, ub0_s),
        (ub1_hbm, ubq_v, uid_v, ub1_s),
        (ub2_hbm, ubq_v, uid_v, ub2_s),
        (ub3_hbm, ubq_v, uid_v, ub3_s)]
    bigs = (big_a, big_b)
    cps = [pltpu.async_copy(stages[0][0].at[stages[0][1]], big_a, sem)]
    for si, (srcr, qv, idv, dst) in enumerate(stages):
        if si + 1 < len(stages):
            nsrc, nqv = stages[si + 1][0], stages[si + 1][1]
            cps.append(pltpu.async_copy(nsrc.at[nqv], bigs[(si + 1) % 2], sem))
        cps[si].wait()
        buf = bigs[si % 2]
        for c in range(NG):
            sl = pl.ds(c * LANES, LANES)
            rows = c * LANES + lane
            dst[sl] = plsc.load_gather(
                buf, [rows, jnp.bitwise_and(idv[sl], c127)])

    def _exp(x):
        t = x * jnp.float32(1.4426950408889634)
        n = t.astype(jnp.int32)
        g = (t - n.astype(jnp.float32)) * jnp.float32(0.6931471805599453)
        p = one + g * jnp.float32(1 / 7.0)
        for r in (6.0, 5.0, 4.0, 3.0, 2.0, 1.0):
            p = one + g * jnp.float32(1 / r) * p
        scale = lax.bitcast_convert_type(
            lax.shift_left(n + 127, jnp.full((LANES,), 23, jnp.int32)),
            jnp.float32)
        return p * scale

    u_bufs = (u_v0, u_v1)
    v_bufs = (v_v0, v_v1)
    pend = [pltpu.async_copy(ue_hbm.at[ueq_v.at[pl.ds(0, CH)]], u_v0, sem),
            pltpu.async_copy(ie_hbm.at[ieq_v.at[pl.ds(0, CH)]], v_v0, sem)]
    for ch in range(BPW // CH):
        u_v = u_bufs[ch % 2]
        v_v = v_bufs[ch % 2]
        pend[0].wait()
        pend[1].wait()
        if ch + 1 < BPW // CH:
            nsl = pl.ds((ch + 1) * CH, CH)
            pend = [pltpu.async_copy(ue_hbm.at[ueq_v.at[nsl]],
                                     u_bufs[(ch + 1) % 2], sem),
                    pltpu.async_copy(ie_hbm.at[ieq_v.at[nsl]],
                                     v_bufs[(ch + 1) % 2], sem)]
        for g in range(CH // LANES):
            e0 = ch * CH + g * LANES
            sl = pl.ds(e0, LANES)
            rows = e0 + lane
            jvec = g * LANES + lane
            ur16 = jnp.bitwise_and(uid_v[sl], 15) * D
            vr16 = jnp.bitwise_and(iid_v[sl], 15) * D

            def dot_step(d4, acc):
                d = d4 * 4
                for s in range(4):
                    ut = ur16 + (d + s)
                    vt = vr16 + (d + s)
                    ug = plsc.load_gather(
                        u_v, [jvec, lax.shift_right_logical(ut, 7),
                              jnp.bitwise_and(ut, c127)])
                    vg = plsc.load_gather(
                        v_v, [jvec, lax.shift_right_logical(vt, 7),
                              jnp.bitwise_and(vt, c127)])
                    acc = acc + ug * vg
                return acc

            y = lax.fori_loop(0, D // 4, dot_step,
                              jnp.zeros((LANES,), jnp.float32))
            y = y + ib_s[sl]

            cut = ub0_s[sl]
            d_prev = one / (one + _exp(y - cut))
            plsc.store_scatter(out_v,
                               [rows, jnp.zeros((LANES,), jnp.int32)], d_prev)
            for k, ub_s in ((1, ub1_s), (2, ub2_s), (3, ub3_s)):
                cut = cut + _exp(ub_s[sl])
                d_k = one / (one + _exp(y - cut))
                plsc.store_scatter(
                    out_v, [rows, jnp.full((LANES,), k, jnp.int32)],
                    d_k - d_prev)
                d_prev = d_k
            plsc.store_scatter(
                out_v, [rows, jnp.full((LANES,), L - 1, jnp.int32)],
                one - d_prev)

    pltpu.sync_copy(out_v, out_hbm.at[pl.ds(base, BPW)])


@functools.partial(
    pl.kernel,
    mesh=plsc.VectorSubcoreMesh(core_axis_name="c", subcore_axis_name="s"),
    out_type=jax.ShapeDtypeStruct((B, L), jnp.float32),
    compiler_params=pltpu.CompilerParams(
        needs_layout_passes=False, use_tc_tiling_on_sc=True),
    scratch_types=[
        pltpu.VMEM((BPW,), jnp.int32),
        pltpu.VMEM((BPW,), jnp.int32),
        pltpu.VMEM((BPW,), jnp.int32),
        pltpu.VMEM((BPW,), jnp.int32),
        pltpu.VMEM((BPW,), jnp.int32),
        pltpu.VMEM((BPW,), jnp.int32),
        pltpu.VMEM((CH, 8, 128), jnp.float32),
        pltpu.VMEM((CH, 8, 128), jnp.float32),
        pltpu.VMEM((CH, 8, 128), jnp.float32),
        pltpu.VMEM((CH, 8, 128), jnp.float32),
        pltpu.VMEM((BPW, 128), jnp.float32),
        pltpu.VMEM((BPW, 128), jnp.float32),
        pltpu.VMEM((BPW,), jnp.float32),
        pltpu.VMEM((BPW,), jnp.float32),
        pltpu.VMEM((BPW,), jnp.float32),
        pltpu.VMEM((BPW,), jnp.float32),
        pltpu.VMEM((BPW,), jnp.float32),
        pltpu.VMEM((BPW, L), jnp.float32),
        pltpu.SemaphoreType.DMA,
    ],
)
def _ordrec_sc(*args):
    _body(*args)


def _colpad(col):
    pad = (-col.shape[0]) % 128
    return jnp.pad(col, (0, pad)).reshape(-1, 128)


def kernel(user_ids, item_ids, user_embeddings, item_embeddings,
           item_biases, user_betas):
    ue3 = user_embeddings.reshape(-1, 8, 128)
    ie3 = item_embeddings.reshape(-1, 8, 128)
    ib2 = _colpad(item_biases[:, 0])
    ub = [_colpad(user_betas[:, k]) for k in range(L - 1)]
    return _ordrec_sc(user_ids.astype(jnp.int32), item_ids.astype(jnp.int32),
                      ue3, ie3, ib2, ub[0], ub[1], ub[2], ub[3])


# speedup vs baseline: 1.4908x; 1.1010x over previous
import functools

import jax
import jax.numpy as jnp
from jax import lax
from jax.experimental import pallas as pl
from jax.experimental.pallas import tpu as pltpu
from jax.experimental.pallas import tpu_sc as plsc

B = 4096
D = 64
L = 5
NC = 2
NS = 16
NW = NC * NS
BPW = B // NW
CH = 128
LANES = 16
NG = BPW // LANES


def _body(uid_hbm, iid_hbm, ue_hbm, ie_hbm, ib_hbm,
          ub0_hbm, ub1_hbm, ub2_hbm, ub3_hbm, out_hbm,
          uid_v, iid_v, ueq_v, ieq_v, ibq_v, ubq_v,
          u_v0, u_v1, v_v0, v_v1, big_a, big_b, ib_s, ub0_s, ub1_s, ub2_s,
          ub3_s, out_v, sem):
    wid = lax.axis_index("s") * NC + lax.axis_index("c")
    base = wid * BPW

    pltpu.sync_copy(uid_hbm.at[pl.ds(base, BPW)], uid_v)
    pltpu.sync_copy(iid_hbm.at[pl.ds(base, BPW)], iid_v)

    for c in range(NG):
        sl = pl.ds(c * LANES, LANES)
        ueq_v[sl] = uid_v[sl]
        ieq_v[sl] = iid_v[sl]
        ubq_v[sl] = lax.shift_right_logical(uid_v[sl], 7)
        ibq_v[sl] = lax.shift_right_logical(iid_v[sl], 7)

    lane = lax.iota(jnp.int32, LANES)
    one = jnp.float32(1.0)
    c127 = jnp.full((LANES,), 127, jnp.int32)

    stages = [
        (ib_hbm, ibq_v, iid_v, ib_s),
        (ub0_hbm, ubq_v, uid_v, ub0_s),
        (ub1_hbm, ubq_v, uid_v, ub1_s),
        (ub2_hbm, ubq_v, uid_v, ub2_s),
        (ub3_hbm, ubq_v, uid_v, ub3_s)]
    bigs = (big_a, big_b)
    cps = [pltpu.async_copy(stages[0][0].at[stages[0][1]], big_a, sem)]
    for si, (srcr, qv, idv, dst) in enumerate(stages):
        if si + 1 < len(stages):
            nsrc, nqv = stages[si + 1][0], stages[si + 1][1]
            cps.append(pltpu.async_copy(nsrc.at[nqv], bigs[(si + 1) % 2], sem))
        cps[si].wait()
        buf = bigs[si % 2]
        for c in range(NG):
            sl = pl.ds(c * LANES, LANES)
            rows = c * LANES + lane
            dst[sl] = plsc.load_gather(
                buf, [rows, jnp.bitwise_and(idv[sl], c127)])

    def _exp(x):
        t = x * jnp.float32(1.4426950408889634)
        n = t.astype(jnp.int32)
        g = (t - n.astype(jnp.float32)) * jnp.float32(0.6931471805599453)
        p = one + g * jnp.float32(1 / 7.0)
        for r in (6.0, 5.0, 4.0, 3.0, 2.0, 1.0):
            p = one + g * jnp.float32(1 / r) * p
        scale = lax.bitcast_convert_type(
            lax.shift_left(n + 127, jnp.full((LANES,), 23, jnp.int32)),
            jnp.float32)
        return p * scale

    u_bufs = (u_v0, u_v1)
    v_bufs = (v_v0, v_v1)
    pend = [pltpu.async_copy(ue_hbm.at[ueq_v.at[pl.ds(0, CH)]], u_v0, sem),
            pltpu.async_copy(ie_hbm.at[ieq_v.at[pl.ds(0, CH)]], v_v0, sem)]
    for ch in range(BPW // CH):
        u_v = u_bufs[ch % 2]
        v_v = v_bufs[ch % 2]
        pend[0].wait()
        pend[1].wait()
        if ch + 1 < BPW // CH:
            nsl = pl.ds((ch + 1) * CH, CH)
            pend = [pltpu.async_copy(ue_hbm.at[ueq_v.at[nsl]],
                                     u_bufs[(ch + 1) % 2], sem),
                    pltpu.async_copy(ie_hbm.at[ieq_v.at[nsl]],
                                     v_bufs[(ch + 1) % 2], sem)]
        for g in range(CH // LANES):
            e0 = ch * CH + g * LANES
            sl = pl.ds(e0, LANES)
            rows = e0 + lane
            jvec = g * LANES + lane
            def dot_step(d4, acc):
                d = d4 * 4
                for s in range(4):
                    col = jnp.full((LANES,), 0, jnp.int32) + (d + s)
                    ug = plsc.load_gather(u_v, [jvec, col])
                    vg = plsc.load_gather(v_v, [jvec, col])
                    acc = acc + ug * vg
                return acc

            y = lax.fori_loop(0, D // 4, dot_step,
                              jnp.zeros((LANES,), jnp.float32))
            y = y + ib_s[sl]

            cut = ub0_s[sl]
            d_prev = one / (one + _exp(y - cut))
            plsc.store_scatter(out_v,
                               [rows, jnp.zeros((LANES,), jnp.int32)], d_prev)
            for k, ub_s in ((1, ub1_s), (2, ub2_s), (3, ub3_s)):
                cut = cut + _exp(ub_s[sl])
                d_k = one / (one + _exp(y - cut))
                plsc.store_scatter(
                    out_v, [rows, jnp.full((LANES,), k, jnp.int32)],
                    d_k - d_prev)
                d_prev = d_k
            plsc.store_scatter(
                out_v, [rows, jnp.full((LANES,), L - 1, jnp.int32)],
                one - d_prev)

    pltpu.sync_copy(out_v, out_hbm.at[pl.ds(base, BPW)])


@functools.partial(
    pl.kernel,
    mesh=plsc.VectorSubcoreMesh(core_axis_name="c", subcore_axis_name="s"),
    out_type=jax.ShapeDtypeStruct((B, L), jnp.float32),
    compiler_params=pltpu.CompilerParams(
        needs_layout_passes=False, use_tc_tiling_on_sc=True),
    scratch_types=[
        pltpu.VMEM((BPW,), jnp.int32),
        pltpu.VMEM((BPW,), jnp.int32),
        pltpu.VMEM((BPW,), jnp.int32),
        pltpu.VMEM((BPW,), jnp.int32),
        pltpu.VMEM((BPW,), jnp.int32),
        pltpu.VMEM((BPW,), jnp.int32),
        pltpu.VMEM((CH, 128), jnp.float32),
        pltpu.VMEM((CH, 128), jnp.float32),
        pltpu.VMEM((CH, 128), jnp.float32),
        pltpu.VMEM((CH, 128), jnp.float32),
        pltpu.VMEM((BPW, 128), jnp.float32),
        pltpu.VMEM((BPW, 128), jnp.float32),
        pltpu.VMEM((BPW,), jnp.float32),
        pltpu.VMEM((BPW,), jnp.float32),
        pltpu.VMEM((BPW,), jnp.float32),
        pltpu.VMEM((BPW,), jnp.float32),
        pltpu.VMEM((BPW,), jnp.float32),
        pltpu.VMEM((BPW, L), jnp.float32),
        pltpu.SemaphoreType.DMA,
    ],
)
def _ordrec_sc(*args):
    _body(*args)


def _colpad(col):
    pad = (-col.shape[0]) % 128
    return jnp.pad(col, (0, pad)).reshape(-1, 128)


def kernel(user_ids, item_ids, user_embeddings, item_embeddings,
           item_biases, user_betas):
    ue3 = jnp.pad(user_embeddings, ((0, 0), (0, 64)))
    ie3 = jnp.pad(item_embeddings, ((0, 0), (0, 64)))
    ib2 = _colpad(item_biases[:, 0])
    ub = [_colpad(user_betas[:, k]) for k in range(L - 1)]
    return _ordrec_sc(user_ids.astype(jnp.int32), item_ids.astype(jnp.int32),
                      ue3, ie3, ib2, ub[0], ub[1], ub[2], ub[3])
